# baseline (speedup 1.0000x reference)
import functools
import jax
import jax.numpy as jnp
from jax import lax
from jax.experimental import pallas as pl
from jax.experimental.pallas import tpu as pltpu
from jax.experimental.pallas import tpu_sc as plsc

D = 64
L = 200
NC = 2
NS = 16
NW = NC * NS

CT = 512
CG = CT // 128


def _body(ids_hbm, seg_hbm, tok_hbm, ps_hbm, out_hbm,
          idx0, seg0, tok0, idx1, seg1, tok1,
          semL0, semP0, semT0, semW0, semL1, semP1, semT1, semW1):
    wid = lax.axis_index("s") * NC + lax.axis_index("c")
    n_tok = ids_hbm.shape[0] * 128
    nt = n_tok // NW
    n_chunks = nt // CT
    w_row0 = wid * (nt // 128)

    iota16 = lax.iota(jnp.int32, 16)
    sets = ((idx0, seg0, tok0, semL0, semP0, semT0, semW0),
            (idx1, seg1, tok1, semL1, semP1, semT1, semW1))

    def rowc(c):
        return w_row0 + lax.min(c, n_chunks - 1) * CG

    def start_L(c, s):
        idx_r, seg_r, sem = sets[s][0], sets[s][1], sets[s][3]
        r = rowc(c)
        pltpu.async_copy(ids_hbm.at[pl.ds(r, CG)], idx_r, sem)
        pltpu.async_copy(seg_hbm.at[pl.ds(r, CG)], seg_r, sem)

    def wait_L(s):
        idx_r, seg_r, sem = sets[s][0], sets[s][1], sets[s][3]
        pltpu.make_async_copy(ids_hbm.at[pl.ds(0, CG)], idx_r, sem).wait()
        pltpu.make_async_copy(seg_hbm.at[pl.ds(0, CG)], seg_r, sem).wait()

    def compute_X(c, s):
        seg_r = sets[s][1]
        r = rowc(c)
        for g in range(CG):
            for u in range(8):
                fb = (r + g) * 128 + u * 16
                lv = lax.rem(iota16 + fb, L)
                sl = pl.ds(u * 16, 16)
                seg_r[g, sl] = seg_r[g, sl] * L + lv

    def start_P(s):
        seg_r, tok_r, sem = sets[s][1], sets[s][2], sets[s][4]
        for g in range(CG):
            pltpu.async_copy(ps_hbm.at[seg_r.at[g]],
                             tok_r.at[pl.ds(g * 128, 128)], sem)

    def wait_P(s):
        tok_r, sem = sets[s][2], sets[s][4]
        for g in range(CG):
            pltpu.make_async_copy(ps_hbm.at[pl.ds(0, 128)],
                                  tok_r.at[pl.ds(g * 128, 128)], sem).wait()

    def start_T(s):
        idx_r, tok_r, sem = sets[s][0], sets[s][2], sets[s][5]
        for g in range(CG):
            pltpu.async_copy(tok_hbm.at[idx_r.at[g]],
                             tok_r.at[pl.ds(g * 128, 128)], sem, add=True)

    def wait_T(s):
        tok_r, sem = sets[s][2], sets[s][5]
        for g in range(CG):
            pltpu.make_async_copy(tok_hbm.at[pl.ds(0, 128)],
                                  tok_r.at[pl.ds(g * 128, 128)], sem).wait()

    def start_W(c, s):
        tok_r, sem = sets[s][2], sets[s][6]
        pltpu.async_copy(tok_r, out_hbm.at[pl.ds((w_row0 + c * CG) * 128, CT)],
                         sem)

    def wait_W(s):
        tok_r, sem = sets[s][2], sets[s][6]
        pltpu.make_async_copy(tok_r, out_hbm.at[pl.ds(0, CT)], sem).wait()

    start_L(0, 0)
    start_L(1, 1)
    wait_L(0); compute_X(0, 0); start_P(0)
    wait_L(1); compute_X(1, 1)
    wait_P(0); start_T(0)
    start_P(1)
    wait_T(0); start_W(0, 0); start_L(2, 0)
    wait_L(0); compute_X(2, 0)
    wait_P(1); start_T(1)
    wait_W(0); start_P(0)
    wait_T(1); start_W(1, 1); start_L(3, 1)

    @pl.loop(2, n_chunks, step=2)
    def _steady(c):
        wait_L(1); compute_X(c + 1, 1)
        wait_P(0); start_T(0)
        wait_W(1); start_P(1)
        wait_T(0); start_W(c, 0); start_L(c + 2, 0)
        wait_L(0); compute_X(c + 2, 0)
        wait_P(1); start_T(1)
        wait_W(0); start_P(0)
        wait_T(1); start_W(c + 1, 1); start_L(c + 3, 1)

    wait_P(0)
    wait_L(1)
    wait_W(1)


@jax.jit
def _sc_call(ids2d, seg2d, token_table, posseg):
    n_tok = ids2d.shape[0] * 128
    mesh = plsc.VectorSubcoreMesh(core_axis_name="c", subcore_axis_name="s")
    f = pl.kernel(
        _body,
        out_type=jax.ShapeDtypeStruct((n_tok, D), jnp.float32),
        mesh=mesh,
        compiler_params=pltpu.CompilerParams(use_tc_tiling_on_sc=False),
        scratch_types=[
            pltpu.VMEM((CG, 128), jnp.int32),
            pltpu.VMEM((CG, 128), jnp.int32),
            pltpu.VMEM((CT, D), jnp.float32),
            pltpu.VMEM((CG, 128), jnp.int32),
            pltpu.VMEM((CG, 128), jnp.int32),
            pltpu.VMEM((CT, D), jnp.float32),
            pltpu.SemaphoreType.DMA,
            pltpu.SemaphoreType.DMA,
            pltpu.SemaphoreType.DMA,
            pltpu.SemaphoreType.DMA,
            pltpu.SemaphoreType.DMA,
            pltpu.SemaphoreType.DMA,
            pltpu.SemaphoreType.DMA,
            pltpu.SemaphoreType.DMA,
        ],
    )
    return f(ids2d, seg2d, token_table, posseg)


def kernel(input_ids, segment_ids, token_table, seg_table, pe):
    B_, L_ = input_ids.shape
    N = B_ * L_
    ids2d = input_ids.reshape(N // 128, 128).astype(jnp.int32)
    seg2d = segment_ids.reshape(N // 128, 128).astype(jnp.int32)
    posseg = (seg_table[:, None, :] + pe[0, :L_][None, :, :]).reshape(3 * L_, D)
    out = _sc_call(ids2d, seg2d, token_table, posseg)
    return out.reshape(B_, L_, D)
